# baseline (speedup 1.0000x reference)
import functools
import math

import jax
import jax.numpy as jnp
from jax import lax
from jax.experimental import pallas as pl
from jax.experimental.pallas import tpu as pltpu
from jax.experimental.pallas import tpu_sc as plsc

TEMP = 0.05
_KB = 1024
_LOG2E = math.log2(math.e)


def _sc_gather(targets_i32, f0, f1, f2):
    B = targets_i32.shape[0]
    D = f0.shape[1]
    info = plsc.get_sparse_core_info()
    nw = info.num_cores * info.num_subcores
    bpw = B // nw
    mesh = plsc.VectorSubcoreMesh(core_axis_name="c", subcore_axis_name="s")

    @functools.partial(
        pl.kernel, mesh=mesh,
        out_type=[jax.ShapeDtypeStruct((B, D), jnp.float32)] * 3,
        scratch_types=[
            pltpu.VMEM((bpw,), jnp.int32),
            pltpu.VMEM((bpw, D), jnp.float32),
            pltpu.SemaphoreType.DMA,
        ],
    )
    def gk(t_hbm, f0_hbm, f1_hbm, f2_hbm, o0, o1, o2, idx_v, rows_v, sem):
        wid = lax.axis_index("s") * info.num_cores + lax.axis_index("c")
        base = wid * bpw
        pltpu.sync_copy(t_hbm.at[pl.ds(base, bpw)], idx_v)
        for f_hbm, o_hbm in ((f0_hbm, o0), (f1_hbm, o1), (f2_hbm, o2)):
            pltpu.async_copy(f_hbm.at[idx_v], rows_v, sem).wait()
            pltpu.sync_copy(rows_v, o_hbm.at[pl.ds(base, bpw)])

    return gk(targets_i32, f0, f1, f2)


def _norm_rows(x):
    n = jnp.sqrt(jnp.sum(x * x, axis=1, keepdims=True))
    return x / jnp.maximum(n, 1e-12)


def _main_body(x_ref, g0_ref, g1_ref, g2_ref, f0_ref, f1_ref, f2_ref,
               out_ref, xsb_ref, s_ref, *, K, nkb):
    B = x_ref.shape[1]
    D = x_ref.shape[2]
    k = pl.program_id(0)
    n_pad = nkb * _KB - K

    @pl.when(k == 0)
    def _init():
        for i in range(3):
            xsb_ref[i] = (_norm_rows(x_ref[i])
                          * (_LOG2E / TEMP)).astype(jnp.bfloat16)
        s_ref[...] = jnp.zeros_like(s_ref)

    col0 = k * _KB
    row_id = lax.broadcasted_iota(jnp.int32, (_KB, D), 0) + col0
    for i, f_ref in enumerate((f0_ref, f1_ref, f2_ref)):
        fb = jnp.where(row_id < K, f_ref[...], 0.0).astype(jnp.bfloat16)
        l2 = lax.dot_general(xsb_ref[i], fb, (((1,), (1,)), ((), ())),
                             preferred_element_type=jnp.float32)
        e = jnp.exp2(l2.astype(jnp.bfloat16))
        s_part = e[:, :128]
        for c in range(1, _KB // 128):
            s_part = s_part + e[:, c * 128:(c + 1) * 128]
        s_ref[i] += s_part.astype(jnp.float32)

    @pl.when(k == nkb - 1)
    def _fini():
        ce = 0.0
        xn = [_norm_rows(x_ref[i]) for i in range(3)]
        for i, g_ref in enumerate((g0_ref, g1_ref, g2_ref)):
            s_row = jnp.sum(s_ref[i], axis=1, keepdims=True) - float(n_pad)
            tgt = jnp.sum(xn[i] * g_ref[...], axis=1,
                          keepdims=True) * (1.0 / TEMP)
            ce += jnp.sum(jnp.log(s_row) - tgt) / B
        x1, x2 = xn[1], xn[2]
        m1 = jnp.max(x1, axis=1, keepdims=True)
        m2 = jnp.max(x2, axis=1, keepdims=True)
        e1 = jnp.exp(x1 - m1)
        e2 = jnp.exp(x2 - m2)
        z1 = jnp.sum(e1, axis=1, keepdims=True)
        z2 = jnp.sum(e2, axis=1, keepdims=True)
        p1 = e1 / z1
        p2 = e2 / z2
        lp1 = (x1 - m1) - jnp.log(z1)
        lp2 = (x2 - m2) - jnp.log(z2)
        lm = jnp.log((p1 + p2) * 0.5)
        kl1 = jnp.sum(p1 * (lp1 - lm))
        kl2 = jnp.sum(p2 * (lp2 - lm))
        out_ref[0, 0] = ce + 0.5 * (kl1 + kl2)


def _run(inputs, targets, f0, f1, f2):
    _, B, D = inputs.shape
    K = f0.shape[0]
    nkb = pl.cdiv(K, _KB)
    t1d = targets.astype(jnp.int32).reshape(B)
    g0, g1, g2 = _sc_gather(t1d, f0, f1, f2)

    body = functools.partial(_main_body, K=K, nkb=nkb)
    out = pl.pallas_call(
        body,
        grid=(nkb,),
        in_specs=[
            pl.BlockSpec((3, B, D), lambda k: (0, 0, 0)),
            pl.BlockSpec((B, D), lambda k: (0, 0)),
            pl.BlockSpec((B, D), lambda k: (0, 0)),
            pl.BlockSpec((B, D), lambda k: (0, 0)),
            pl.BlockSpec((_KB, D), lambda k: (k, 0)),
            pl.BlockSpec((_KB, D), lambda k: (k, 0)),
            pl.BlockSpec((_KB, D), lambda k: (k, 0)),
        ],
        out_specs=pl.BlockSpec(memory_space=pltpu.SMEM),
        out_shape=jax.ShapeDtypeStruct((1, 1), jnp.float32),
        scratch_shapes=[
            pltpu.VMEM((3, B, D), jnp.bfloat16),
            pltpu.VMEM((3, B, 128), jnp.float32),
        ],
        compiler_params=pltpu.CompilerParams(
            dimension_semantics=("arbitrary",)),
    )(inputs, g0, g1, g2, f0, f1, f2)
    return out[0, 0]


def kernel(inputs, targets, predict_features, global_p1_features,
           global_p2_features):
    return _run(inputs, targets, predict_features, global_p1_features,
                global_p2_features)


# speedup vs baseline: 1.0041x; 1.0041x over previous
import functools
import math

import jax
import jax.numpy as jnp
from jax import lax
from jax.experimental import pallas as pl
from jax.experimental.pallas import tpu as pltpu
from jax.experimental.pallas import tpu_sc as plsc

TEMP = 0.05
_KB = 1024
_LOG2E = math.log2(math.e)


def _sc_gather(targets_i32, f0, f1, f2):
    B = targets_i32.shape[0]
    D = f0.shape[1]
    info = plsc.get_sparse_core_info()
    nw = info.num_cores * info.num_subcores
    bpw = B // nw
    mesh = plsc.VectorSubcoreMesh(core_axis_name="c", subcore_axis_name="s")

    @functools.partial(
        pl.kernel, mesh=mesh,
        out_type=[jax.ShapeDtypeStruct((B, D), jnp.float32)] * 3,
        scratch_types=[
            pltpu.VMEM((bpw,), jnp.int32),
            pltpu.VMEM((bpw, D), jnp.float32),
            pltpu.SemaphoreType.DMA,
        ],
    )
    def gk(t_hbm, f0_hbm, f1_hbm, f2_hbm, o0, o1, o2, idx_v, rows_v, sem):
        wid = lax.axis_index("s") * info.num_cores + lax.axis_index("c")
        base = wid * bpw
        pltpu.sync_copy(t_hbm.at[pl.ds(base, bpw)], idx_v)
        for f_hbm, o_hbm in ((f0_hbm, o0), (f1_hbm, o1), (f2_hbm, o2)):
            pltpu.async_copy(f_hbm.at[idx_v], rows_v, sem).wait()
            pltpu.sync_copy(rows_v, o_hbm.at[pl.ds(base, bpw)])

    return gk(targets_i32, f0, f1, f2)


def _norm_rows(x):
    n = jnp.sqrt(jnp.sum(x * x, axis=1, keepdims=True))
    return x / jnp.maximum(n, 1e-12)


def _main_body(x_ref, g0_ref, g1_ref, g2_ref, f0_ref, f1_ref, f2_ref,
               out_ref, xsb_ref, s_ref, *, K, nkb):
    B = x_ref.shape[1]
    D = x_ref.shape[2]
    k = pl.program_id(0)
    n_pad = nkb * _KB - K

    @pl.when(k == 0)
    def _init():
        for i in range(3):
            xsb_ref[i] = (_norm_rows(x_ref[i])
                          * (_LOG2E / TEMP)).astype(jnp.bfloat16)
        s_ref[...] = jnp.zeros_like(s_ref)

    col0 = k * _KB
    row_id = lax.broadcasted_iota(jnp.int32, (_KB, D), 0) + col0
    for i, f_ref in enumerate((f0_ref, f1_ref, f2_ref)):
        fb = jnp.where(row_id < K, f_ref[...], 0.0).astype(jnp.bfloat16)
        l2 = lax.dot_general(xsb_ref[i], fb, (((1,), (1,)), ((), ())),
                             preferred_element_type=jnp.float32)
        e = jnp.exp2(l2)
        s_part = e[:, :128]
        for c in range(1, _KB // 128):
            s_part = s_part + e[:, c * 128:(c + 1) * 128]
        s_ref[i] += s_part

    @pl.when(k == nkb - 1)
    def _fini():
        ce = 0.0
        xn = [_norm_rows(x_ref[i]) for i in range(3)]
        for i, g_ref in enumerate((g0_ref, g1_ref, g2_ref)):
            s_row = jnp.sum(s_ref[i], axis=1, keepdims=True) - float(n_pad)
            tgt = jnp.sum(xn[i] * g_ref[...], axis=1,
                          keepdims=True) * (1.0 / TEMP)
            ce += jnp.sum(jnp.log(s_row) - tgt) / B
        x1, x2 = xn[1], xn[2]
        m1 = jnp.max(x1, axis=1, keepdims=True)
        m2 = jnp.max(x2, axis=1, keepdims=True)
        e1 = jnp.exp(x1 - m1)
        e2 = jnp.exp(x2 - m2)
        z1 = jnp.sum(e1, axis=1, keepdims=True)
        z2 = jnp.sum(e2, axis=1, keepdims=True)
        p1 = e1 / z1
        p2 = e2 / z2
        lp1 = (x1 - m1) - jnp.log(z1)
        lp2 = (x2 - m2) - jnp.log(z2)
        lm = jnp.log((p1 + p2) * 0.5)
        kl1 = jnp.sum(p1 * (lp1 - lm))
        kl2 = jnp.sum(p2 * (lp2 - lm))
        out_ref[0, 0] = ce + 0.5 * (kl1 + kl2)


def _run(inputs, targets, f0, f1, f2):
    _, B, D = inputs.shape
    K = f0.shape[0]
    nkb = pl.cdiv(K, _KB)
    t1d = targets.astype(jnp.int32).reshape(B)
    g0, g1, g2 = _sc_gather(t1d, f0, f1, f2)

    body = functools.partial(_main_body, K=K, nkb=nkb)
    out = pl.pallas_call(
        body,
        grid=(nkb,),
        in_specs=[
            pl.BlockSpec((3, B, D), lambda k: (0, 0, 0)),
            pl.BlockSpec((B, D), lambda k: (0, 0)),
            pl.BlockSpec((B, D), lambda k: (0, 0)),
            pl.BlockSpec((B, D), lambda k: (0, 0)),
            pl.BlockSpec((_KB, D), lambda k: (k, 0)),
            pl.BlockSpec((_KB, D), lambda k: (k, 0)),
            pl.BlockSpec((_KB, D), lambda k: (k, 0)),
        ],
        out_specs=pl.BlockSpec(memory_space=pltpu.SMEM),
        out_shape=jax.ShapeDtypeStruct((1, 1), jnp.float32),
        scratch_shapes=[
            pltpu.VMEM((3, B, D), jnp.bfloat16),
            pltpu.VMEM((3, B, 128), jnp.float32),
        ],
        compiler_params=pltpu.CompilerParams(
            dimension_semantics=("arbitrary",)),
    )(inputs, g0, g1, g2, f0, f1, f2)
    return out[0, 0]


def kernel(inputs, targets, predict_features, global_p1_features,
           global_p2_features):
    return _run(inputs, targets, predict_features, global_p1_features,
                global_p2_features)


# speedup vs baseline: 1.0052x; 1.0011x over previous
import functools
import math

import jax
import jax.numpy as jnp
from jax import lax
from jax.experimental import pallas as pl
from jax.experimental.pallas import tpu as pltpu
from jax.experimental.pallas import tpu_sc as plsc

TEMP = 0.05
_KB = 1024
_LOG2E = math.log2(math.e)


def _sc_gather(targets_i32, f0, f1, f2):
    B = targets_i32.shape[0]
    D = f0.shape[1]
    info = plsc.get_sparse_core_info()
    nw = info.num_cores * info.num_subcores
    bpw = B // nw
    mesh = plsc.VectorSubcoreMesh(core_axis_name="c", subcore_axis_name="s")

    @functools.partial(
        pl.kernel, mesh=mesh,
        out_type=[jax.ShapeDtypeStruct((B, D), jnp.float32)] * 3,
        scratch_types=[
            pltpu.VMEM((bpw,), jnp.int32),
            pltpu.VMEM((bpw, D), jnp.float32),
            pltpu.SemaphoreType.DMA,
        ],
    )
    def gk(t_hbm, f0_hbm, f1_hbm, f2_hbm, o0, o1, o2, idx_v, rows_v, sem):
        wid = lax.axis_index("s") * info.num_cores + lax.axis_index("c")
        base = wid * bpw
        pltpu.sync_copy(t_hbm.at[pl.ds(base, bpw)], idx_v)
        for f_hbm, o_hbm in ((f0_hbm, o0), (f1_hbm, o1), (f2_hbm, o2)):
            pltpu.async_copy(f_hbm.at[idx_v], rows_v, sem).wait()
            pltpu.sync_copy(rows_v, o_hbm.at[pl.ds(base, bpw)])

    return gk(targets_i32, f0, f1, f2)


def _norm_rows(x):
    n = jnp.sqrt(jnp.sum(x * x, axis=1, keepdims=True))
    return x / jnp.maximum(n, 1e-12)


def _main_body(x_ref, g0_ref, g1_ref, g2_ref, f0_ref, f1_ref, f2_ref,
               out_ref, xsb_ref, s_ref, m_ref, *, K, nkb):
    B = x_ref.shape[1]
    D = x_ref.shape[2]
    k = pl.program_id(0)
    n_pad = nkb * _KB - K

    @pl.when(k == 0)
    def _init():
        for i in range(3):
            xsb_ref[i] = (_norm_rows(x_ref[i])
                          * (_LOG2E / TEMP)).astype(jnp.bfloat16)
        s_ref[...] = jnp.zeros_like(s_ref)
        m_ref[...] = jnp.ones_like(m_ref)

    @pl.when(k == nkb - 1)
    def _mask():
        row_id = lax.broadcasted_iota(jnp.int32, (_KB, D), 0) + k * _KB
        m_ref[...] = jnp.where(row_id < K, 1.0, 0.0)

    for i, f_ref in enumerate((f0_ref, f1_ref, f2_ref)):
        fb = (f_ref[...] * m_ref[...]).astype(jnp.bfloat16)
        l2 = lax.dot_general(xsb_ref[i], fb, (((1,), (1,)), ((), ())),
                             preferred_element_type=jnp.float32)
        e = jnp.exp2(l2)
        s_part = e[:, :128]
        for c in range(1, _KB // 128):
            s_part = s_part + e[:, c * 128:(c + 1) * 128]
        s_ref[i] += s_part

    @pl.when(k == nkb - 1)
    def _fini():
        ce = 0.0
        xn = [_norm_rows(x_ref[i]) for i in range(3)]
        for i, g_ref in enumerate((g0_ref, g1_ref, g2_ref)):
            s_row = jnp.sum(s_ref[i], axis=1, keepdims=True) - float(n_pad)
            tgt = jnp.sum(xn[i] * g_ref[...], axis=1,
                          keepdims=True) * (1.0 / TEMP)
            ce += jnp.sum(jnp.log(s_row) - tgt) / B
        x1, x2 = xn[1], xn[2]
        m1 = jnp.max(x1, axis=1, keepdims=True)
        m2 = jnp.max(x2, axis=1, keepdims=True)
        e1 = jnp.exp(x1 - m1)
        e2 = jnp.exp(x2 - m2)
        z1 = jnp.sum(e1, axis=1, keepdims=True)
        z2 = jnp.sum(e2, axis=1, keepdims=True)
        p1 = e1 / z1
        p2 = e2 / z2
        lp1 = (x1 - m1) - jnp.log(z1)
        lp2 = (x2 - m2) - jnp.log(z2)
        lm = jnp.log((p1 + p2) * 0.5)
        kl1 = jnp.sum(p1 * (lp1 - lm))
        kl2 = jnp.sum(p2 * (lp2 - lm))
        out_ref[0, 0] = ce + 0.5 * (kl1 + kl2)


def _run(inputs, targets, f0, f1, f2):
    _, B, D = inputs.shape
    K = f0.shape[0]
    nkb = pl.cdiv(K, _KB)
    t1d = targets.astype(jnp.int32).reshape(B)
    g0, g1, g2 = _sc_gather(t1d, f0, f1, f2)

    body = functools.partial(_main_body, K=K, nkb=nkb)
    out = pl.pallas_call(
        body,
        grid=(nkb,),
        in_specs=[
            pl.BlockSpec((3, B, D), lambda k: (0, 0, 0)),
            pl.BlockSpec((B, D), lambda k: (0, 0)),
            pl.BlockSpec((B, D), lambda k: (0, 0)),
            pl.BlockSpec((B, D), lambda k: (0, 0)),
            pl.BlockSpec((_KB, D), lambda k: (k, 0)),
            pl.BlockSpec((_KB, D), lambda k: (k, 0)),
            pl.BlockSpec((_KB, D), lambda k: (k, 0)),
        ],
        out_specs=pl.BlockSpec(memory_space=pltpu.SMEM),
        out_shape=jax.ShapeDtypeStruct((1, 1), jnp.float32),
        scratch_shapes=[
            pltpu.VMEM((3, B, D), jnp.bfloat16),
            pltpu.VMEM((3, B, 128), jnp.float32),
            pltpu.VMEM((_KB, D), jnp.float32),
        ],
        compiler_params=pltpu.CompilerParams(
            dimension_semantics=("arbitrary",)),
    )(inputs, g0, g1, g2, f0, f1, f2)
    return out[0, 0]


def kernel(inputs, targets, predict_features, global_p1_features,
           global_p2_features):
    return _run(inputs, targets, predict_features, global_p1_features,
                global_p2_features)
